# baseline (speedup 1.0000x reference)
import jax
import jax.numpy as jnp
from jax.experimental import pallas as pl
from jax.experimental.pallas import tpu as pltpu

N = 4096
R = 4
INDIM = 128
OUTDIM = 16

BM = 128


def _mrgcn_kernel(x_ref, w2_ref, a_ref, o_ref, xw_ref):
    @pl.when(pl.program_id(0) == 0)
    def _():
        y = jnp.dot(x_ref[...], w2_ref[...],
                    preferred_element_type=jnp.float32)
        for r in range(R):
            xw_ref[r * N:(r + 1) * N, :] = (
                y[:, r * OUTDIM:(r + 1) * OUTDIM].astype(jnp.bfloat16))

    @pl.when(pl.program_id(0) > 0)
    def _():
        acc = jnp.dot(a_ref[...].astype(jnp.bfloat16), xw_ref[...],
                      preferred_element_type=jnp.float32)
        o_ref[...] = jnp.maximum(acc, 0.0)


def kernel(X, A, W):
    W2 = W.reshape(R, INDIM, OUTDIM).transpose(1, 0, 2).reshape(
        INDIM, R * OUTDIM)
    return pl.pallas_call(
        _mrgcn_kernel,
        grid=(N // BM + 1,),
        in_specs=[
            pl.BlockSpec((N, INDIM), lambda m: (0, 0)),
            pl.BlockSpec((INDIM, R * OUTDIM), lambda m: (0, 0)),
            pl.BlockSpec((BM, R * N), lambda m: (jnp.maximum(m - 1, 0), 0)),
        ],
        out_specs=pl.BlockSpec(
            (BM, OUTDIM), lambda m: (jnp.maximum(m - 1, 0), 0)),
        out_shape=jax.ShapeDtypeStruct((N, OUTDIM), jnp.float32),
        scratch_shapes=[pltpu.VMEM((R * N, OUTDIM), jnp.bfloat16)],
    )(X, W2, A)


# speedup vs baseline: 1.0129x; 1.0129x over previous
import jax
import jax.numpy as jnp
from jax.experimental import pallas as pl
from jax.experimental.pallas import tpu as pltpu

N = 4096
R = 4
INDIM = 128
OUTDIM = 16

BM = 128


def _mrgcn_kernel(x_ref, w2_ref, a_ref, o_ref, xw_ref):
    @pl.when(pl.program_id(0) == 0)
    def _():
        y = jnp.dot(x_ref[...], w2_ref[...],
                    preferred_element_type=jnp.float32)
        for r in range(R):
            xw_ref[r * N:(r + 1) * N, :] = (
                y[:, r * OUTDIM:(r + 1) * OUTDIM].astype(jnp.bfloat16))

    acc = jnp.dot(a_ref[...].astype(jnp.bfloat16), xw_ref[...],
                  preferred_element_type=jnp.float32)
    o_ref[...] = jnp.maximum(acc, 0.0)


def kernel(X, A, W):
    W2 = W.reshape(R, INDIM, OUTDIM).transpose(1, 0, 2).reshape(
        INDIM, R * OUTDIM)
    return pl.pallas_call(
        _mrgcn_kernel,
        grid=(N // BM,),
        in_specs=[
            pl.BlockSpec((N, INDIM), lambda m: (0, 0)),
            pl.BlockSpec((INDIM, R * OUTDIM), lambda m: (0, 0)),
            pl.BlockSpec((BM, R * N), lambda m: (m, 0)),
        ],
        out_specs=pl.BlockSpec((BM, OUTDIM), lambda m: (m, 0)),
        out_shape=jax.ShapeDtypeStruct((N, OUTDIM), jnp.float32),
        scratch_shapes=[pltpu.VMEM((R * N, OUTDIM), jnp.bfloat16)],
    )(X, W2, A)
